# baseline (speedup 1.0000x reference)
import functools

import jax
import jax.numpy as jnp
from jax import lax
from jax.experimental import pallas as pl
from jax.experimental.pallas import tpu as pltpu
from jax.experimental.pallas import tpu_sc as plsc

MARGIN = 8.0
BATCH = 16384
EMB_DIM = 64
NUM_ENT = 1000000
NUM_REL = 1000
NUM_CORES = 2
NUM_SUBCORES = 16
NUM_TILES = NUM_CORES * NUM_SUBCORES
NEVENTS = 2 * BATCH
EPT = NEVENTS // NUM_TILES
LANES = 16
PANEL = 128
EVBATCH = 128
BPW = BATCH // NUM_TILES
K2CHUNK = 64

_mesh = plsc.VectorSubcoreMesh(core_axis_name="c", subcore_axis_name="s")
_params = pltpu.CompilerParams(
    needs_layout_passes=False,
    use_tc_tiling_on_sc=True,
    disable_bounds_checks=True,
)


RING = 12


@functools.partial(
    pl.kernel,
    mesh=_mesh,
    out_type=jax.ShapeDtypeStruct((NEVENTS, 2 * EMB_DIM), jnp.float32),
    compiler_params=_params,
    scratch_types=[
        pltpu.VMEM((EPT,), jnp.int32),
        pltpu.VMEM((EPT,), jnp.int32),
        pltpu.VMEM((EPT,), jnp.int32),
        pltpu.VMEM((EPT // PANEL, PANEL), jnp.int32),
        pltpu.VMEM((2 * LANES,), jnp.int32),
        pltpu.VMEM((EMB_DIM, RING * PANEL), jnp.float32),
        pltpu.VMEM((EVBATCH, 2 * EMB_DIM), jnp.float32),
        pltpu.SemaphoreType.DMA,
        pltpu.SemaphoreType.DMA,
    ],
)
def _sweep_kernel(pan_hbm, col_hbm, fpan_hbm, slot_hbm, pro_hbm,
                  entT_hbm, scratch_hbm,
                  pan_i, col_i, fp_i, slot_i, pro_i, ring_v, ev_v,
                  sem, sem_sc):
    tid = lax.axis_index("s") * NUM_CORES + lax.axis_index("c")
    base = tid * EPT
    lane_iota = lax.iota(jnp.int32, LANES)

    pltpu.sync_copy(pan_hbm.at[pl.ds(base, EPT)], pan_i)
    pltpu.sync_copy(col_hbm.at[pl.ds(base, EPT)], col_i)
    pltpu.sync_copy(fpan_hbm.at[pl.ds(base, EPT)], fp_i)
    pltpu.sync_copy(slot_hbm.at[pl.ds(tid * (EPT // PANEL), EPT // PANEL), :],
                    slot_i)
    pltpu.sync_copy(pro_hbm.at[pl.ds(tid * 2 * LANES, 2 * LANES)], pro_i)

    def fire(panel, ofs):
        src = pl.ds(pl.multiple_of(panel * PANEL, PANEL), PANEL)
        dst = pl.ds(pl.multiple_of(ofs * PANEL, PANEL), PANEL)
        pltpu.async_copy(entT_hbm.at[:, src], ring_v.at[:, dst], sem)

    def drain_one():
        pltpu.make_async_copy(entT_hbm.at[:, pl.ds(0, PANEL)],
                              ring_v.at[:, pl.ds(0, PANEL)], sem).wait()

    pro16 = pro_i[pl.ds(0, LANES)]
    for j in range(RING - 1):
        fire(pro16[j], j)

    def blk_body(b, carry):
        prev_pan, gpos = carry
        s = pl.ds(b * LANES, LANES)
        pan16 = pan_i[s]
        col16 = col_i[s]
        fp16 = fp_i[s]
        for lane in range(LANES):
            p = pan16[lane]
            is_sw = p != prev_pan
            gpos = lax.select(is_sw, lax.rem(gpos + 1, RING), gpos)

            @pl.when(is_sw)
            def _():
                fire(fp16[lane], lax.rem(gpos + RING - 1, RING))
                drain_one()

            prev_pan = p
            bc = col16[lane] + gpos * PANEL
            ev = b % (EVBATCH // LANES) * LANES + lane
            for k in range(EMB_DIM // LANES):
                g = plsc.load_gather(
                    ring_v, [k * LANES + lane_iota,
                             jnp.broadcast_to(bc, (LANES,))]
                )
                ev_v[ev, pl.ds(k * LANES, LANES)] = g

        @pl.when(b % (EVBATCH // LANES) == EVBATCH // LANES - 1)
        def _():
            batch = b // (EVBATCH // LANES)
            pltpu.async_copy(ev_v, scratch_hbm.at[slot_i.at[batch]],
                             sem_sc).wait()

        return prev_pan, gpos

    lax.fori_loop(0, EPT // LANES, blk_body,
                  (jnp.int32(-1), jnp.int32(-1)))
    for _ in range(RING - 1):
        drain_one()


@functools.partial(
    pl.kernel,
    mesh=_mesh,
    out_type=jax.ShapeDtypeStruct((BATCH,), jnp.float32),
    compiler_params=_params,
    scratch_types=[
        pltpu.VMEM((BPW,), jnp.int32),
        pltpu.VMEM((EMB_DIM, NUM_REL), jnp.float32),
        pltpu.VMEM((2 * K2CHUNK, 2 * EMB_DIM), jnp.float32),
        pltpu.VMEM((2 * K2CHUNK, 2 * EMB_DIM), jnp.float32),
        pltpu.VMEM((BPW,), jnp.float32),
        pltpu.SemaphoreType.DMA,
        pltpu.SemaphoreType.DMA,
    ],
)
def _score_kernel(r_hbm, relT_hbm, scratch_hbm, out_hbm,
                  r_i, rel_v, ev_a, ev_b, o_v, sem_a, sem_b):
    wid = lax.axis_index("s") * NUM_CORES + lax.axis_index("c")
    base = wid * BPW
    lane_iota = lax.iota(jnp.int32, LANES)

    pltpu.sync_copy(r_hbm.at[pl.ds(base, BPW)], r_i)
    pltpu.sync_copy(relT_hbm, rel_v)

    nchunk = BPW // K2CHUNK
    bufs = [(ev_a, sem_a), (ev_b, sem_b)]

    def fetch(ci):
        buf, sem = bufs[ci % 2]
        return pltpu.async_copy(
            scratch_hbm.at[pl.ds(2 * base + ci * 2 * K2CHUNK, 2 * K2CHUNK), :],
            buf, sem,
        )

    pending = fetch(0)
    for ci in range(nchunk):
        nxt = fetch(ci + 1) if ci + 1 < nchunk else None
        pending.wait()
        pending = nxt
        ev_v = bufs[ci % 2][0]

        def blk_body(i, inner, ci=ci, ev_v=ev_v):
            rows = i * LANES + lane_iota
            r16 = r_i[pl.ds(ci * K2CHUNK + i * LANES, LANES)]
            acc = jnp.zeros((LANES,), jnp.float32)
            for d in range(EMB_DIM):
                cols = jnp.full((LANES,), d, jnp.int32)
                hv = plsc.load_gather(ev_v, [2 * rows, cols])
                tv = plsc.load_gather(ev_v, [2 * rows + 1, cols])
                rv = plsc.load_gather(rel_v, [cols, r16])
                acc = acc + jnp.abs(hv + rv - tv)
            o_v[pl.ds(ci * K2CHUNK + i * LANES, LANES)] = MARGIN - acc
            return inner

        lax.fori_loop(0, K2CHUNK // LANES, blk_body, 0)

    pltpu.sync_copy(o_v, out_hbm.at[pl.ds(base, BPW)])


def kernel(sample, ent_emb, relation_embedding):
    h = sample[:, 0]
    r = sample[:, 1]
    t = sample[:, 2]
    ents = jnp.concatenate([h, t])
    slots = jnp.concatenate(
        [2 * jnp.arange(BATCH, dtype=jnp.int32),
         2 * jnp.arange(BATCH, dtype=jnp.int32) + 1]
    )
    e_s, slot_sorted = lax.sort((ents, slots), num_keys=1)
    pan_s = e_s >> 7
    col_s = e_s & 127
    slot_s = slot_sorted.reshape(NEVENTS // PANEL, PANEL)

    idx = jnp.arange(NEVENTS, dtype=jnp.int32)
    flag = (pan_s != jnp.roll(pan_s, 1)) | (idx % EPT == 0)
    sid = jnp.cumsum(flag.astype(jnp.int32)) - 1
    pan_of_sid = jnp.zeros((NEVENTS,), jnp.int32).at[sid].set(pan_s)
    tgt = jnp.minimum(sid + (RING - 1), NEVENTS - 1)
    fpan = pan_of_sid[tgt]
    sid0 = sid[:: EPT]
    pro = pan_of_sid[
        jnp.minimum(sid0[:, None]
                    + jnp.arange(2 * LANES, dtype=jnp.int32)[None, :],
                    NEVENTS - 1)
    ].reshape(-1)

    scratch = _sweep_kernel(pan_s, col_s, fpan, slot_s, pro, ent_emb.T)
    out = _score_kernel(r, relation_embedding.T, scratch)
    return out[:, None]


# speedup vs baseline: 1.6518x; 1.6518x over previous
import functools

import jax
import jax.numpy as jnp
from jax import lax
from jax.experimental import pallas as pl
from jax.experimental.pallas import tpu as pltpu
from jax.experimental.pallas import tpu_sc as plsc

MARGIN = 8.0
BATCH = 16384
EMB_DIM = 64
NUM_ENT = 1000000
NUM_REL = 1000
NUM_CORES = 2
NUM_SUBCORES = 16
NUM_TILES = NUM_CORES * NUM_SUBCORES
NEVENTS = 2 * BATCH
EPT = NEVENTS // NUM_TILES
LANES = 16
PANEL = 128
EVBATCH = 128
BPW = BATCH // NUM_TILES
K2CHUNK = 64

_mesh = plsc.VectorSubcoreMesh(core_axis_name="c", subcore_axis_name="s")
_params = pltpu.CompilerParams(
    needs_layout_passes=False,
    use_tc_tiling_on_sc=True,
    disable_bounds_checks=True,
)


RING = 12


@functools.partial(
    pl.kernel,
    mesh=_mesh,
    out_type=jax.ShapeDtypeStruct((NEVENTS, 2 * EMB_DIM), jnp.float32),
    compiler_params=_params,
    scratch_types=[
        pltpu.VMEM((EPT,), jnp.int32),
        pltpu.VMEM((EPT,), jnp.int32),
        pltpu.VMEM((EPT,), jnp.int32),
        pltpu.VMEM((EPT // PANEL, PANEL), jnp.int32),
        pltpu.VMEM((EMB_DIM, RING * PANEL), jnp.float32),
        pltpu.VMEM((EVBATCH, 2 * EMB_DIM), jnp.float32),
        pltpu.SemaphoreType.DMA,
        pltpu.SemaphoreType.DMA,
    ],
)
def _sweep_kernel(pan_hbm, col_hbm, slot_hbm, entT_hbm, scratch_hbm,
                  pan_i, col_i, dl_i, slot_i, ring_v, ev_v,
                  sem, sem_sc):
    tid = lax.axis_index("s") * NUM_CORES + lax.axis_index("c")
    base = tid * EPT
    lane_iota = lax.iota(jnp.int32, LANES)
    lane0 = lane_iota == 0

    pltpu.sync_copy(pan_hbm.at[pl.ds(base, EPT)], pan_i)
    pltpu.sync_copy(col_hbm.at[pl.ds(base, EPT)], col_i)
    pltpu.sync_copy(slot_hbm.at[pl.ds(tid * (EPT // PANEL), EPT // PANEL), :],
                    slot_i)

    def pre_body(b, carry):
        prev_pan, s = carry
        pan16 = pan_i[pl.ds(b * LANES, LANES)]
        for lane in range(LANES):
            p = pan16[lane]
            is_sw = p != prev_pan
            s = lax.select(is_sw, s + 1, s)
            plsc.store_scatter(dl_i, [jnp.broadcast_to(s, (LANES,))],
                               jnp.broadcast_to(p, (LANES,)),
                               mask=lane0 & is_sw)
            prev_pan = p
        return prev_pan, s

    _, last_g = lax.fori_loop(0, EPT // LANES, pre_body,
                              (jnp.int32(-1), jnp.int32(-1)))

    def dpanel(g):
        return plsc.load_gather(dl_i, [jnp.broadcast_to(g, (LANES,))])[0]

    def fire(panel, ofs):
        src = pl.ds(pl.multiple_of(panel * PANEL, PANEL), PANEL)
        dst = pl.ds(pl.multiple_of(ofs * PANEL, PANEL), PANEL)
        pltpu.async_copy(entT_hbm.at[:, src], ring_v.at[:, dst], sem)

    def drain_one():
        pltpu.make_async_copy(entT_hbm.at[:, pl.ds(0, PANEL)],
                              ring_v.at[:, pl.ds(0, PANEL)], sem).wait()

    for j in range(RING - 1):
        fire(dpanel(lax.min(jnp.int32(j), last_g)), j)

    def blk_body(b, carry):
        prev_pan, g, gpos = carry
        s = pl.ds(b * LANES, LANES)
        pan16 = pan_i[s]
        col16 = col_i[s]
        for lane in range(LANES):
            p = pan16[lane]
            is_sw = p != prev_pan
            g = lax.select(is_sw, g + 1, g)
            gpos = lax.select(is_sw, lax.rem(gpos + 1, RING), gpos)

            @pl.when(is_sw)
            def _():
                fire(dpanel(lax.min(g + RING - 1, last_g)),
                     lax.rem(gpos + RING - 1, RING))
                drain_one()

            prev_pan = p
            bc = col16[lane] + gpos * PANEL
            ev = b % (EVBATCH // LANES) * LANES + lane
            for k in range(EMB_DIM // LANES):
                vals = plsc.load_gather(
                    ring_v, [k * LANES + lane_iota,
                             jnp.broadcast_to(bc, (LANES,))]
                )
                ev_v[ev, pl.ds(k * LANES, LANES)] = vals

        @pl.when(b % (EVBATCH // LANES) == EVBATCH // LANES - 1)
        def _():
            batch = b // (EVBATCH // LANES)
            pltpu.async_copy(ev_v, scratch_hbm.at[slot_i.at[batch]],
                             sem_sc).wait()

        return prev_pan, g, gpos

    lax.fori_loop(0, EPT // LANES, blk_body,
                  (jnp.int32(-1), jnp.int32(-1), jnp.int32(-1)))
    for _ in range(RING - 1):
        drain_one()


@functools.partial(
    pl.kernel,
    mesh=_mesh,
    out_type=jax.ShapeDtypeStruct((BATCH,), jnp.float32),
    compiler_params=_params,
    scratch_types=[
        pltpu.VMEM((BPW,), jnp.int32),
        pltpu.VMEM((EMB_DIM, NUM_REL), jnp.float32),
        pltpu.VMEM((2 * K2CHUNK, 2 * EMB_DIM), jnp.float32),
        pltpu.VMEM((2 * K2CHUNK, 2 * EMB_DIM), jnp.float32),
        pltpu.VMEM((BPW,), jnp.float32),
        pltpu.SemaphoreType.DMA,
        pltpu.SemaphoreType.DMA,
    ],
)
def _score_kernel(r_hbm, relT_hbm, scratch_hbm, out_hbm,
                  r_i, rel_v, ev_a, ev_b, o_v, sem_a, sem_b):
    wid = lax.axis_index("s") * NUM_CORES + lax.axis_index("c")
    base = wid * BPW
    lane_iota = lax.iota(jnp.int32, LANES)

    pltpu.sync_copy(r_hbm.at[pl.ds(base, BPW)], r_i)
    pltpu.sync_copy(relT_hbm, rel_v)

    nchunk = BPW // K2CHUNK
    bufs = [(ev_a, sem_a), (ev_b, sem_b)]

    def fetch(ci):
        buf, sem = bufs[ci % 2]
        return pltpu.async_copy(
            scratch_hbm.at[pl.ds(2 * base + ci * 2 * K2CHUNK, 2 * K2CHUNK), :],
            buf, sem,
        )

    pending = fetch(0)
    for ci in range(nchunk):
        nxt = fetch(ci + 1) if ci + 1 < nchunk else None
        pending.wait()
        pending = nxt
        ev_v = bufs[ci % 2][0]

        def blk_body(i, inner, ci=ci, ev_v=ev_v):
            rows = i * LANES + lane_iota
            r16 = r_i[pl.ds(ci * K2CHUNK + i * LANES, LANES)]
            acc = jnp.zeros((LANES,), jnp.float32)
            for d in range(EMB_DIM):
                cols = jnp.full((LANES,), d, jnp.int32)
                hv = plsc.load_gather(ev_v, [2 * rows, cols])
                tv = plsc.load_gather(ev_v, [2 * rows + 1, cols])
                rv = plsc.load_gather(rel_v, [cols, r16])
                acc = acc + jnp.abs(hv + rv - tv)
            o_v[pl.ds(ci * K2CHUNK + i * LANES, LANES)] = MARGIN - acc
            return inner

        lax.fori_loop(0, K2CHUNK // LANES, blk_body, 0)

    pltpu.sync_copy(o_v, out_hbm.at[pl.ds(base, BPW)])


def kernel(sample, ent_emb, relation_embedding):
    h = sample[:, 0]
    r = sample[:, 1]
    t = sample[:, 2]
    ents = jnp.concatenate([h, t])
    slots = jnp.concatenate(
        [2 * jnp.arange(BATCH, dtype=jnp.int32),
         2 * jnp.arange(BATCH, dtype=jnp.int32) + 1]
    )
    e_s, slot_sorted = lax.sort((ents, slots), num_keys=1)
    pan_s = e_s >> 7
    col_s = e_s & 127
    slot_s = slot_sorted.reshape(NEVENTS // PANEL, PANEL)

    scratch = _sweep_kernel(pan_s, col_s, slot_s, ent_emb.T)
    out = _score_kernel(r, relation_embedding.T, scratch)
    return out[:, None]
